# baseline (speedup 1.0000x reference)
import math

import jax
import jax.numpy as jnp
from jax.experimental import pallas as pl
from jax.experimental.pallas import tpu as pltpu
from jax.experimental.pallas import tpu_sc as plsc

_PAD = 0
_CONF = 0.9
_N = 2048
_V = 32000
_W = 1280
_GRID = _V // _W
_SLABS = _W // 128
_LAG = 16

_L1 = _CONF
_L0 = (1.0 - _CONF) / (_V - 2)
_C = _L1 * math.log(_L1) + (_V - 1) * _L0 * math.log(_L0)


def _sum_body(m_ref, yp_ref, s_out_ref, npad_ref, s_ref):
    j = pl.program_id(0)

    @pl.when(j == 0)
    def _():
        s_ref[...] = jnp.zeros((_N, 128), jnp.float32)

    part = yp_ref[:, 0:128]
    for c in range(1, _SLABS):
        part = part + yp_ref[:, c * 128:(c + 1) * 128]
    s_ref[...] += part

    @pl.when(j == _GRID - 1)
    def _():
        m = m_ref[...]
        s_out_ref[0, 0] = jnp.sum(s_ref[...] * m)
        npad_ref[0, 0] = jnp.sum(m).astype(jnp.int32)


def _masked_sum(yp, mrow):
    return pl.pallas_call(
        _sum_body,
        grid=(_GRID,),
        in_specs=[
            pl.BlockSpec((_N, 1), lambda j: (0, 0)),
            pl.BlockSpec((_N, _W), lambda j: (0, j)),
        ],
        out_specs=[
            pl.BlockSpec(memory_space=pltpu.SMEM),
            pl.BlockSpec(memory_space=pltpu.SMEM),
        ],
        out_shape=[
            jax.ShapeDtypeStruct((1, 1), jnp.float32),
            jax.ShapeDtypeStruct((1, 1), jnp.int32),
        ],
        scratch_shapes=[pltpu.VMEM((_N, 128), jnp.float32)],
    )(mrow, yp)


def _sc_gather(yp, calign):
    mesh = plsc.ScalarSubcoreMesh(axis_name="core", num_cores=2)
    half = _N // 2

    @pl.kernel(
        out_type=jax.ShapeDtypeStruct((8 * _N, 128), jnp.float32),
        mesh=mesh,
        scratch_types=[
            pltpu.SMEM((_N,), jnp.int32),
            pltpu.SemaphoreType.DMA,
            pltpu.SemaphoreType.DMA,
        ],
    )
    def _k(yp_hbm, c_hbm, o_hbm, c_smem, idx_sem, sem):
        core = jax.lax.axis_index("core")
        base = core * half
        pltpu.async_copy(
            c_hbm.at[pl.ds(base, half)], c_smem.at[pl.ds(base, half)], idx_sem
        ).wait()

        @pl.loop(0, half)
        def _(i):
            n = base + i
            r0 = (n // 8) * 8
            pltpu.async_copy(
                yp_hbm.at[pl.ds(r0, 8), pl.ds(pl.multiple_of(c_smem[n], 128), 128)],
                o_hbm.at[pl.ds(n * 8, 8)],
                sem,
            )

            @pl.when(i >= _LAG)
            def _():
                pltpu.make_async_copy(
                    yp_hbm.at[pl.ds(0, 8), pl.ds(0, 128)],
                    o_hbm.at[pl.ds(0, 8)],
                    sem,
                ).wait()

        @pl.loop(0, _LAG)
        def _(i):
            pltpu.make_async_copy(
                yp_hbm.at[pl.ds(0, 8), pl.ds(0, 128)],
                o_hbm.at[pl.ds(0, 8)],
                sem,
            ).wait()

    return _k(yp, calign)


def _combine_body(g_ref, lane_ref, s_ref, npad_ref, loss_ref, npad_out):
    lanes = jax.lax.broadcasted_iota(jnp.int32, (8 * _N, 128), 1)
    sel = jnp.where(lanes == lane_ref[...], g_ref[...], 0.0)
    gsum = jnp.sum(sel)
    npad_out[0, 0] = npad_ref[0, 0]
    loss_ref[0, 0] = (
        npad_ref[0, 0].astype(jnp.float32) * _C
        - _L0 * s_ref[0, 0]
        - (_L1 - _L0) * gsum
    )


def kernel(y_pred, y_true):
    yp = y_pred.reshape(_N, _V)
    yt = y_true.reshape(_N, 1)
    nonpad = yt != _PAD
    mrow = nonpad.astype(jnp.float32)
    calign = (yt // 128 * 128).reshape(_N)
    sub = jnp.arange(_N, dtype=jnp.int32).reshape(_N, 1) % 8
    sub8 = jax.lax.broadcasted_iota(jnp.int32, (_N, 8), 1)
    lane8 = jnp.where((sub8 == sub) & nonpad, yt % 128, -1)
    lane2 = lane8.reshape(8 * _N, 1)

    s, npad = _masked_sum(yp, mrow)
    g = _sc_gather(yp, calign)

    loss, npad_out = pl.pallas_call(
        _combine_body,
        in_specs=[
            pl.BlockSpec((8 * _N, 128), lambda: (0, 0)),
            pl.BlockSpec((8 * _N, 1), lambda: (0, 0)),
            pl.BlockSpec(memory_space=pltpu.SMEM),
            pl.BlockSpec(memory_space=pltpu.SMEM),
        ],
        out_specs=[
            pl.BlockSpec(memory_space=pltpu.SMEM),
            pl.BlockSpec(memory_space=pltpu.SMEM),
        ],
        out_shape=[
            jax.ShapeDtypeStruct((1, 1), jnp.float32),
            jax.ShapeDtypeStruct((1, 1), jnp.int32),
        ],
    )(g, lane2, s, npad)
    return (loss[0, 0], npad_out[0, 0])
